# Initial kernel scaffold; baseline (speedup 1.0000x reference)
import jax
import jax.numpy as jnp
from jax.experimental import pallas as pl


def kernel(x, edge_index, W1l, b1l, W1r, W2l, b2l, W2r, Wm1, bm1, Wm2, bm2):
    raise NotImplementedError("write your pallas kernel here")


# speedup vs baseline: 4.0925x; 4.0925x over previous
import functools

import jax
import jax.numpy as jnp
from jax import lax
from jax.experimental import pallas as pl
from jax.experimental.pallas import tpu as pltpu
from jax.experimental.pallas import tpu_sc as plsc

N = 10000
E = 320000
D = 128
H = 128

NC = 2
NS = 16
NW = NC * NS
CH = 80
EPT = E // NW
NCH = EPT // CH
NP = 10240
RPT = NP // NS
LANES = 16


def _mesh():
    return plsc.VectorSubcoreMesh(
        core_axis_name="c", subcore_axis_name="s", num_cores=NC, num_subcores=NS
    )


@functools.partial(
    pl.kernel,
    out_type=jax.ShapeDtypeStruct((NC, NP, D), jnp.float32),
    mesh=_mesh(),
    scratch_types=[
        pltpu.VMEM((CH,), jnp.int32),
        pltpu.VMEM((CH, D), jnp.float32),
        pltpu.VMEM((CH, D), jnp.float32),
        pltpu.VMEM_SHARED((NP, D), jnp.float32),
        pltpu.SemaphoreType.DMA,
    ],
)
def _sc_cnt(dst_hbm, zeros_hbm, ones_hbm, cnt_out, didx, rows, ones, cntS, sem):
    cid = lax.axis_index("c")
    sid = lax.axis_index("s")
    wid = cid * NS + sid

    rowid16 = lax.iota(jnp.int32, LANES)
    pltpu.sync_copy(zeros_hbm, rows)
    pltpu.sync_copy(ones_hbm, ones)

    start = sid * RPT
    for z in range(RPT // CH):
        zbase = start + z * CH
        for k in range(CH // LANES):
            didx[pl.ds(LANES * k, LANES)] = rowid16 + (zbase + LANES * k)
        pltpu.sync_copy(rows, cntS.at[didx])
    plsc.subcore_barrier()

    ebase = wid * EPT

    @pl.loop(0, NCH)
    def _chunk(c):
        base = ebase + c * CH
        pltpu.sync_copy(dst_hbm.at[pl.ds(base, CH)], didx)
        pltpu.sync_copy(ones, cntS.at[didx], add=True)

    plsc.subcore_barrier()
    for z in range(RPT // CH):
        zbase = start + z * CH
        for k in range(CH // LANES):
            didx[pl.ds(LANES * k, LANES)] = rowid16 + (zbase + LANES * k)
        pltpu.async_copy(cntS.at[didx], rows, sem).wait()
        pltpu.sync_copy(rows, cnt_out.at[cid, pl.ds(zbase, CH)])


@functools.partial(
    pl.kernel,
    out_type=jax.ShapeDtypeStruct((NC, NP, D), jnp.float32),
    mesh=_mesh(),
    scratch_types=[
        pltpu.VMEM((CH,), jnp.int32),
        pltpu.VMEM((CH,), jnp.int32),
        pltpu.VMEM((CH, D), jnp.float32),
        pltpu.VMEM_SHARED((NP, D), jnp.float32),
        pltpu.SemaphoreType.DMA,
    ],
)
def _sc_agg(x_hbm, src_hbm, dst_hbm, zeros_hbm, agg_out, sidx, didx, rows, aggS, sem):
    cid = lax.axis_index("c")
    sid = lax.axis_index("s")
    wid = cid * NS + sid

    rowid16 = lax.iota(jnp.int32, LANES)

    pltpu.sync_copy(zeros_hbm, rows)

    start = sid * RPT
    for z in range(RPT // CH):
        zbase = start + z * CH
        for k in range(CH // LANES):
            didx[pl.ds(LANES * k, LANES)] = rowid16 + (zbase + LANES * k)
        pltpu.sync_copy(rows, aggS.at[didx])
    plsc.subcore_barrier()

    ebase = wid * EPT

    @pl.loop(0, NCH)
    def _chunk(c):
        base = ebase + c * CH
        pltpu.sync_copy(src_hbm.at[pl.ds(base, CH)], sidx)
        pltpu.sync_copy(dst_hbm.at[pl.ds(base, CH)], didx)
        pltpu.async_copy(x_hbm.at[sidx], rows, sem).wait()
        pltpu.sync_copy(rows, aggS.at[didx], add=True)

    plsc.subcore_barrier()
    for z in range(RPT // CH):
        zbase = start + z * CH
        for k in range(CH // LANES):
            didx[pl.ds(LANES * k, LANES)] = rowid16 + (zbase + LANES * k)
        pltpu.async_copy(aggS.at[didx], rows, sem).wait()
        pltpu.sync_copy(rows, agg_out.at[cid, pl.ds(zbase, CH)])


@functools.partial(
    pl.kernel,
    out_type=jax.ShapeDtypeStruct((E,), jnp.float32),
    mesh=_mesh(),
    scratch_types=[
        pltpu.VMEM((CH,), jnp.int32),
        pltpu.VMEM((CH,), jnp.int32),
        pltpu.VMEM((CH, D), jnp.float32),
        pltpu.VMEM((CH, D), jnp.float32),
        pltpu.VMEM((CH,), jnp.float32),
        pltpu.VMEM((D,), jnp.float32),
        pltpu.VMEM((LANES,), jnp.float32),
        pltpu.SemaphoreType.DMA,
        pltpu.SemaphoreType.DMA,
    ],
)
def _sc_edge(a_hbm, b_hbm, src_hbm, dst_hbm, w_hbm, bm2_hbm, out_hbm,
             sidx, didx, rowsA, rowsB, outb, wbuf, bbuf, semA, semB):
    cid = lax.axis_index("c")
    sid = lax.axis_index("s")
    wid = cid * NS + sid

    pltpu.sync_copy(w_hbm, wbuf)
    pltpu.sync_copy(bm2_hbm, bbuf)
    wv = [wbuf[pl.ds(LANES * j, LANES)] for j in range(D // LANES)]
    b2 = bbuf[...]
    lane = lax.iota(jnp.int32, LANES)

    ebase = wid * EPT

    @pl.loop(0, NCH)
    def _chunk(c):
        base = ebase + c * CH
        pltpu.sync_copy(src_hbm.at[pl.ds(base, CH)], sidx)
        pltpu.sync_copy(dst_hbm.at[pl.ds(base, CH)], didx)
        cpa = pltpu.async_copy(a_hbm.at[sidx], rowsA, semA)
        cpb = pltpu.async_copy(b_hbm.at[didx], rowsB, semB)
        cpa.wait()
        cpb.wait()

        @pl.loop(0, CH // LANES)
        def _grp(g):
            vec = jnp.zeros((LANES,), jnp.float32)
            for e16 in range(LANES):
                e = g * LANES + e16
                acc = None
                for j in range(D // LANES):
                    va = rowsA[e, pl.ds(LANES * j, LANES)]
                    vb = rowsB[e, pl.ds(LANES * j, LANES)]
                    t = jnp.maximum(va + vb, 0.0) * wv[j]
                    acc = t if acc is None else acc + t
                s = acc[0]
                for l in range(1, LANES):
                    s = s + acc[l]
                vec = jnp.where(lane == e16, s, vec)
            vec = vec + b2
            outb[pl.ds(g * LANES, LANES)] = 1.0 / (1.0 + jnp.exp(-vec))

        pltpu.sync_copy(outb, out_hbm.at[pl.ds(base, CH)])


_BN = 1000
_PREC = lax.Precision.HIGHEST


def _tc_layer1_body(aggp, cntp, x, w1l, b1l, w1r, h1_out):
    cnt = cntp[0, :, 0:1] + cntp[1, :, 0:1]
    inv = 1.0 / jnp.maximum(cnt, 1.0)
    mean1 = (aggp[0] + aggp[1]) * inv
    z = (
        jnp.dot(mean1, w1l[...], preferred_element_type=jnp.float32, precision=_PREC)
        + b1l[...]
        + jnp.dot(x[...], w1r[...], preferred_element_type=jnp.float32, precision=_PREC)
    )
    h1_out[...] = jnp.maximum(z, 0.0)


def _tc_layer1(aggp, cntp, x, w1l, b1l, w1r):
    grid = N // _BN
    return pl.pallas_call(
        _tc_layer1_body,
        grid=(grid,),
        in_specs=[
            pl.BlockSpec((NC, _BN, D), lambda i: (0, i, 0)),
            pl.BlockSpec((NC, _BN, D), lambda i: (0, i, 0)),
            pl.BlockSpec((_BN, D), lambda i: (i, 0)),
            pl.BlockSpec((D, H), lambda i: (0, 0)),
            pl.BlockSpec((1, H), lambda i: (0, 0)),
            pl.BlockSpec((D, H), lambda i: (0, 0)),
        ],
        out_specs=pl.BlockSpec((_BN, H), lambda i: (i, 0)),
        out_shape=jax.ShapeDtypeStruct((N, H), jnp.float32),
    )(aggp, cntp, x, w1l, b1l, w1r)


def _tc_layer2_body(aggp, cntp, h1, w2l, b2l, w2r, wm1t, wm1b, bm1, a_out, b_out):
    cnt = cntp[0, :, 0:1] + cntp[1, :, 0:1]
    inv = 1.0 / jnp.maximum(cnt, 1.0)
    mean2 = (aggp[0] + aggp[1]) * inv
    z = (
        jnp.dot(mean2, w2l[...], preferred_element_type=jnp.float32, precision=_PREC)
        + b2l[...]
        + jnp.dot(h1[...], w2r[...], preferred_element_type=jnp.float32, precision=_PREC)
    )
    h2 = jnp.maximum(z, 0.0)
    a_out[...] = (
        jnp.dot(h2, wm1t[...], preferred_element_type=jnp.float32, precision=_PREC)
        + bm1[...]
    )
    b_out[...] = jnp.dot(h2, wm1b[...], preferred_element_type=jnp.float32, precision=_PREC)


def _tc_layer2(aggp, cntp, h1, w2l, b2l, w2r, wm1t, wm1b, bm1):
    grid = N // _BN
    return pl.pallas_call(
        _tc_layer2_body,
        grid=(grid,),
        in_specs=[
            pl.BlockSpec((NC, _BN, D), lambda i: (0, i, 0)),
            pl.BlockSpec((NC, _BN, D), lambda i: (0, i, 0)),
            pl.BlockSpec((_BN, H), lambda i: (i, 0)),
            pl.BlockSpec((H, H), lambda i: (0, 0)),
            pl.BlockSpec((1, H), lambda i: (0, 0)),
            pl.BlockSpec((H, H), lambda i: (0, 0)),
            pl.BlockSpec((H, H), lambda i: (0, 0)),
            pl.BlockSpec((H, H), lambda i: (0, 0)),
            pl.BlockSpec((1, H), lambda i: (0, 0)),
        ],
        out_specs=[
            pl.BlockSpec((_BN, H), lambda i: (i, 0)),
            pl.BlockSpec((_BN, H), lambda i: (i, 0)),
        ],
        out_shape=[
            jax.ShapeDtypeStruct((N, H), jnp.float32),
            jax.ShapeDtypeStruct((N, H), jnp.float32),
        ],
    )(aggp, cntp, h1, w2l, b2l, w2r, wm1t, wm1b, bm1)


def _jnp_sage(x, src, dst, Wl, bl, Wr):
    msg = jnp.take(x, src, axis=0)
    agg = jax.ops.segment_sum(msg, dst, num_segments=N)
    cnt = jax.ops.segment_sum(jnp.ones((src.shape[0],), x.dtype), dst, num_segments=N)
    mean = agg / jnp.clip(cnt, 1.0)[:, None]
    return mean @ Wl + bl + x @ Wr


def kernel(x, edge_index, W1l, b1l, W1r, W2l, b2l, W2r, Wm1, bm1, Wm2, bm2):
    src = edge_index[0]
    dst = edge_index[1]
    zeros_c = jnp.zeros((CH, D), jnp.float32)
    ones_c = jnp.ones((CH, D), jnp.float32)
    cntp = _sc_cnt(dst, zeros_c, ones_c)
    aggp = _sc_agg(x, src, dst, zeros_c)
    h1 = _tc_layer1(aggp, cntp, x, W1l, b1l.reshape(1, H), W1r)
    agg2p = _sc_agg(h1, src, dst, zeros_c)
    a_nodes, b_nodes = _tc_layer2(
        agg2p, cntp, h1, W2l, b2l.reshape(1, H), W2r,
        Wm1[:H], Wm1[H:], bm1.reshape(1, H),
    )
    bm2v = jnp.broadcast_to(bm2, (LANES,))
    return _sc_edge(a_nodes, b_nodes, src, dst, Wm2.reshape(H), bm2v)


# speedup vs baseline: 4.1008x; 1.0020x over previous
import functools

import jax
import jax.numpy as jnp
from jax import lax
from jax.experimental import pallas as pl
from jax.experimental.pallas import tpu as pltpu
from jax.experimental.pallas import tpu_sc as plsc

N = 10000
E = 320000
D = 128
H = 128

NC = 2
NS = 16
NW = NC * NS
CH = 80
EPT = E // NW
NCH = EPT // CH
NP = 10240
RPT = NP // NS
LANES = 16


def _mesh():
    return plsc.VectorSubcoreMesh(
        core_axis_name="c", subcore_axis_name="s", num_cores=NC, num_subcores=NS
    )


@functools.partial(
    pl.kernel,
    out_type=jax.ShapeDtypeStruct((NC, NP, D), jnp.float32),
    mesh=_mesh(),
    scratch_types=[
        pltpu.VMEM((CH,), jnp.int32),
        pltpu.VMEM((CH, D), jnp.float32),
        pltpu.VMEM((CH, D), jnp.float32),
        pltpu.VMEM_SHARED((NP, D), jnp.float32),
        pltpu.SemaphoreType.DMA,
    ],
)
def _sc_cnt(dst_hbm, zeros_hbm, ones_hbm, cnt_out, didx, rows, ones, cntS, sem):
    cid = lax.axis_index("c")
    sid = lax.axis_index("s")
    wid = cid * NS + sid

    rowid16 = lax.iota(jnp.int32, LANES)
    pltpu.sync_copy(zeros_hbm, rows)
    pltpu.sync_copy(ones_hbm, ones)

    start = sid * RPT
    for z in range(RPT // CH):
        zbase = start + z * CH
        for k in range(CH // LANES):
            didx[pl.ds(LANES * k, LANES)] = rowid16 + (zbase + LANES * k)
        pltpu.sync_copy(rows, cntS.at[didx])
    plsc.subcore_barrier()

    ebase = wid * EPT

    @pl.loop(0, NCH)
    def _chunk(c):
        base = ebase + c * CH
        pltpu.sync_copy(dst_hbm.at[pl.ds(base, CH)], didx)
        pltpu.sync_copy(ones, cntS.at[didx], add=True)

    plsc.subcore_barrier()
    for z in range(RPT // CH):
        zbase = start + z * CH
        for k in range(CH // LANES):
            didx[pl.ds(LANES * k, LANES)] = rowid16 + (zbase + LANES * k)
        pltpu.async_copy(cntS.at[didx], rows, sem).wait()
        pltpu.sync_copy(rows, cnt_out.at[cid, pl.ds(zbase, CH)])


@functools.partial(
    pl.kernel,
    out_type=jax.ShapeDtypeStruct((NC, NP, D), jnp.float32),
    mesh=_mesh(),
    scratch_types=[
        pltpu.VMEM((CH,), jnp.int32),
        pltpu.VMEM((CH,), jnp.int32),
        pltpu.VMEM((CH, D), jnp.float32),
        pltpu.VMEM_SHARED((NP, D), jnp.float32),
        pltpu.SemaphoreType.DMA,
    ],
)
def _sc_agg(x_hbm, src_hbm, dst_hbm, zeros_hbm, agg_out, sidx, didx, rows, aggS, sem):
    cid = lax.axis_index("c")
    sid = lax.axis_index("s")
    wid = cid * NS + sid

    rowid16 = lax.iota(jnp.int32, LANES)

    pltpu.sync_copy(zeros_hbm, rows)

    start = sid * RPT
    for z in range(RPT // CH):
        zbase = start + z * CH
        for k in range(CH // LANES):
            didx[pl.ds(LANES * k, LANES)] = rowid16 + (zbase + LANES * k)
        pltpu.sync_copy(rows, aggS.at[didx])
    plsc.subcore_barrier()

    ebase = wid * EPT

    @pl.loop(0, NCH)
    def _chunk(c):
        base = ebase + c * CH
        pltpu.sync_copy(src_hbm.at[pl.ds(base, CH)], sidx)
        pltpu.sync_copy(dst_hbm.at[pl.ds(base, CH)], didx)
        pltpu.async_copy(x_hbm.at[sidx], rows, sem).wait()
        pltpu.sync_copy(rows, aggS.at[didx], add=True)

    plsc.subcore_barrier()
    for z in range(RPT // CH):
        zbase = start + z * CH
        for k in range(CH // LANES):
            didx[pl.ds(LANES * k, LANES)] = rowid16 + (zbase + LANES * k)
        pltpu.async_copy(aggS.at[didx], rows, sem).wait()
        pltpu.sync_copy(rows, agg_out.at[cid, pl.ds(zbase, CH)])


@functools.partial(
    pl.kernel,
    out_type=jax.ShapeDtypeStruct((E * LANES,), jnp.float32),
    mesh=_mesh(),
    scratch_types=[
        pltpu.VMEM((CH,), jnp.int32),
        pltpu.VMEM((CH,), jnp.int32),
        pltpu.VMEM((CH, D), jnp.float32),
        pltpu.VMEM((CH, D), jnp.float32),
        pltpu.VMEM((CH * LANES,), jnp.float32),
        pltpu.VMEM((D,), jnp.float32),
        pltpu.SemaphoreType.DMA,
        pltpu.SemaphoreType.DMA,
    ],
)
def _sc_edge(a_hbm, b_hbm, src_hbm, dst_hbm, w_hbm, out_hbm,
             sidx, didx, rowsA, rowsB, accb, wbuf, semA, semB):
    cid = lax.axis_index("c")
    sid = lax.axis_index("s")
    wid = cid * NS + sid

    pltpu.sync_copy(w_hbm, wbuf)
    wv = [wbuf[pl.ds(LANES * j, LANES)] for j in range(D // LANES)]

    ebase = wid * EPT

    @pl.loop(0, NCH)
    def _chunk(c):
        base = ebase + c * CH
        pltpu.sync_copy(src_hbm.at[pl.ds(base, CH)], sidx)
        pltpu.sync_copy(dst_hbm.at[pl.ds(base, CH)], didx)
        cpa = pltpu.async_copy(a_hbm.at[sidx], rowsA, semA)
        cpb = pltpu.async_copy(b_hbm.at[didx], rowsB, semB)
        cpa.wait()
        cpb.wait()

        @pl.loop(0, CH)
        def _edge(e):
            acc = None
            for j in range(D // LANES):
                va = rowsA[e, pl.ds(LANES * j, LANES)]
                vb = rowsB[e, pl.ds(LANES * j, LANES)]
                t = jnp.maximum(va + vb, 0.0) * wv[j]
                acc = t if acc is None else acc + t
            accb[pl.ds(e * LANES, LANES)] = acc

        pltpu.sync_copy(accb, out_hbm.at[pl.ds(base * LANES, CH * LANES)])


_BN = 1000
_PREC = lax.Precision.HIGHEST


def _tc_layer1_body(aggp, cntp, x, w1l, b1l, w1r, h1_out):
    cnt = cntp[0, :, 0:1] + cntp[1, :, 0:1]
    inv = 1.0 / jnp.maximum(cnt, 1.0)
    mean1 = (aggp[0] + aggp[1]) * inv
    z = (
        jnp.dot(mean1, w1l[...], preferred_element_type=jnp.float32, precision=_PREC)
        + b1l[...]
        + jnp.dot(x[...], w1r[...], preferred_element_type=jnp.float32, precision=_PREC)
    )
    h1_out[...] = jnp.maximum(z, 0.0)


def _tc_layer1(aggp, cntp, x, w1l, b1l, w1r):
    grid = N // _BN
    return pl.pallas_call(
        _tc_layer1_body,
        grid=(grid,),
        in_specs=[
            pl.BlockSpec((NC, _BN, D), lambda i: (0, i, 0)),
            pl.BlockSpec((NC, _BN, D), lambda i: (0, i, 0)),
            pl.BlockSpec((_BN, D), lambda i: (i, 0)),
            pl.BlockSpec((D, H), lambda i: (0, 0)),
            pl.BlockSpec((1, H), lambda i: (0, 0)),
            pl.BlockSpec((D, H), lambda i: (0, 0)),
        ],
        out_specs=pl.BlockSpec((_BN, H), lambda i: (i, 0)),
        out_shape=jax.ShapeDtypeStruct((N, H), jnp.float32),
    )(aggp, cntp, x, w1l, b1l, w1r)


def _tc_layer2_body(aggp, cntp, h1, w2l, b2l, w2r, wm1t, wm1b, bm1, a_out, b_out):
    cnt = cntp[0, :, 0:1] + cntp[1, :, 0:1]
    inv = 1.0 / jnp.maximum(cnt, 1.0)
    mean2 = (aggp[0] + aggp[1]) * inv
    z = (
        jnp.dot(mean2, w2l[...], preferred_element_type=jnp.float32, precision=_PREC)
        + b2l[...]
        + jnp.dot(h1[...], w2r[...], preferred_element_type=jnp.float32, precision=_PREC)
    )
    h2 = jnp.maximum(z, 0.0)
    a_out[...] = (
        jnp.dot(h2, wm1t[...], preferred_element_type=jnp.float32, precision=_PREC)
        + bm1[...]
    )
    b_out[...] = jnp.dot(h2, wm1b[...], preferred_element_type=jnp.float32, precision=_PREC)


def _tc_layer2(aggp, cntp, h1, w2l, b2l, w2r, wm1t, wm1b, bm1):
    grid = N // _BN
    return pl.pallas_call(
        _tc_layer2_body,
        grid=(grid,),
        in_specs=[
            pl.BlockSpec((NC, _BN, D), lambda i: (0, i, 0)),
            pl.BlockSpec((NC, _BN, D), lambda i: (0, i, 0)),
            pl.BlockSpec((_BN, H), lambda i: (i, 0)),
            pl.BlockSpec((H, H), lambda i: (0, 0)),
            pl.BlockSpec((1, H), lambda i: (0, 0)),
            pl.BlockSpec((H, H), lambda i: (0, 0)),
            pl.BlockSpec((H, H), lambda i: (0, 0)),
            pl.BlockSpec((H, H), lambda i: (0, 0)),
            pl.BlockSpec((1, H), lambda i: (0, 0)),
        ],
        out_specs=[
            pl.BlockSpec((_BN, H), lambda i: (i, 0)),
            pl.BlockSpec((_BN, H), lambda i: (i, 0)),
        ],
        out_shape=[
            jax.ShapeDtypeStruct((N, H), jnp.float32),
            jax.ShapeDtypeStruct((N, H), jnp.float32),
        ],
    )(aggp, cntp, h1, w2l, b2l, w2r, wm1t, wm1b, bm1)


def _jnp_sage(x, src, dst, Wl, bl, Wr):
    msg = jnp.take(x, src, axis=0)
    agg = jax.ops.segment_sum(msg, dst, num_segments=N)
    cnt = jax.ops.segment_sum(jnp.ones((src.shape[0],), x.dtype), dst, num_segments=N)
    mean = agg / jnp.clip(cnt, 1.0)[:, None]
    return mean @ Wl + bl + x @ Wr


_PR = E * LANES // 128
_BPR = 4000


def _tc_final_body(p, g, bm2, out):
    s = jnp.dot(p[...], g[...], preferred_element_type=jnp.float32, precision=_PREC)
    out[...] = jax.nn.sigmoid(s + bm2[...])


def _tc_final(p128, g, bm2):
    grid = _PR // _BPR
    return pl.pallas_call(
        _tc_final_body,
        grid=(grid,),
        in_specs=[
            pl.BlockSpec((_BPR, 128), lambda i: (i, 0)),
            pl.BlockSpec((128, 8), lambda i: (0, 0)),
            pl.BlockSpec((1, 1), lambda i: (0, 0)),
        ],
        out_specs=pl.BlockSpec((_BPR, 8), lambda i: (i, 0)),
        out_shape=jax.ShapeDtypeStruct((_PR, 8), jnp.float32),
    )(p128, g, bm2)


def kernel(x, edge_index, W1l, b1l, W1r, W2l, b2l, W2r, Wm1, bm1, Wm2, bm2):
    src = edge_index[0]
    dst = edge_index[1]
    zeros_c = jnp.zeros((CH, D), jnp.float32)
    ones_c = jnp.ones((CH, D), jnp.float32)
    cntp = _sc_cnt(dst, zeros_c, ones_c)
    aggp = _sc_agg(x, src, dst, zeros_c)
    h1 = _tc_layer1(aggp, cntp, x, W1l, b1l.reshape(1, H), W1r)
    agg2p = _sc_agg(h1, src, dst, zeros_c)
    a_nodes, b_nodes = _tc_layer2(
        agg2p, cntp, h1, W2l, b2l.reshape(1, H), W2r,
        Wm1[:H], Wm1[H:], bm1.reshape(1, H),
    )
    p_flat = _sc_edge(a_nodes, b_nodes, src, dst, Wm2.reshape(H))
    g = (jnp.arange(128)[:, None] // LANES == jnp.arange(8)[None, :]).astype(jnp.float32)
    out8 = _tc_final(p_flat.reshape(_PR, 128), g, bm2.reshape(1, 1))
    return out8.reshape(E)


# speedup vs baseline: 5.0548x; 1.2326x over previous
import functools

import jax
import jax.numpy as jnp
from jax import lax
from jax.experimental import pallas as pl
from jax.experimental.pallas import tpu as pltpu
from jax.experimental.pallas import tpu_sc as plsc

N = 10000
E = 320000
D = 128
H = 128

NC = 2
NS = 16
NW = NC * NS
CH = 80
EPT = E // NW
NCH = EPT // CH
NP = 10240
RPT = NP // NS
LANES = 16


def _mesh():
    return plsc.VectorSubcoreMesh(
        core_axis_name="c", subcore_axis_name="s", num_cores=NC, num_subcores=NS
    )


@functools.partial(
    pl.kernel,
    out_type=jax.ShapeDtypeStruct((NC, NP, D), jnp.float32),
    mesh=_mesh(),
    scratch_types=[
        pltpu.VMEM((CH,), jnp.int32),
        pltpu.VMEM((CH, D), jnp.float32),
        pltpu.VMEM((CH, D), jnp.float32),
        pltpu.VMEM_SHARED((NP, D), jnp.float32),
        pltpu.SemaphoreType.DMA,
    ],
)
def _sc_cnt(dst_hbm, zeros_hbm, ones_hbm, cnt_out, didx, rows, ones, cntS, sem):
    cid = lax.axis_index("c")
    sid = lax.axis_index("s")
    wid = cid * NS + sid

    rowid16 = lax.iota(jnp.int32, LANES)
    pltpu.sync_copy(zeros_hbm, rows)
    pltpu.sync_copy(ones_hbm, ones)

    start = sid * RPT
    for z in range(RPT // CH):
        zbase = start + z * CH
        for k in range(CH // LANES):
            didx[pl.ds(LANES * k, LANES)] = rowid16 + (zbase + LANES * k)
        pltpu.sync_copy(rows, cntS.at[didx])
    plsc.subcore_barrier()

    ebase = wid * EPT

    @pl.loop(0, NCH)
    def _chunk(c):
        base = ebase + c * CH
        pltpu.sync_copy(dst_hbm.at[pl.ds(base, CH)], didx)
        pltpu.sync_copy(ones, cntS.at[didx], add=True)

    plsc.subcore_barrier()
    for z in range(RPT // CH):
        zbase = start + z * CH
        for k in range(CH // LANES):
            didx[pl.ds(LANES * k, LANES)] = rowid16 + (zbase + LANES * k)
        pltpu.async_copy(cntS.at[didx], rows, sem).wait()
        pltpu.sync_copy(rows, cnt_out.at[cid, pl.ds(zbase, CH)])


@functools.partial(
    pl.kernel,
    out_type=jax.ShapeDtypeStruct((NC, NP, D), jnp.float32),
    mesh=_mesh(),
    scratch_types=[
        pltpu.VMEM((CH,), jnp.int32),
        pltpu.VMEM((CH,), jnp.int32),
        pltpu.VMEM((CH,), jnp.int32),
        pltpu.VMEM((CH,), jnp.int32),
        pltpu.VMEM((CH, D), jnp.float32),
        pltpu.VMEM((CH, D), jnp.float32),
        pltpu.VMEM_SHARED((NP, D), jnp.float32),
        pltpu.SemaphoreType.DMA,
        pltpu.SemaphoreType.DMA,
    ],
)
def _sc_agg(x_hbm, src_hbm, dst_hbm, zeros_hbm, agg_out,
            sidx0, didx0, sidx1, didx1, rows0, rows1, aggS, sem0, sem1):
    cid = lax.axis_index("c")
    sid = lax.axis_index("s")
    wid = cid * NS + sid

    rowid16 = lax.iota(jnp.int32, LANES)

    pltpu.sync_copy(zeros_hbm, rows0)

    start = sid * RPT
    for z in range(RPT // CH):
        zbase = start + z * CH
        for k in range(CH // LANES):
            didx0[pl.ds(LANES * k, LANES)] = rowid16 + (zbase + LANES * k)
        pltpu.sync_copy(rows0, aggS.at[didx0])
    plsc.subcore_barrier()

    ebase = wid * EPT
    bufs = ((sidx0, didx0, rows0, sem0), (sidx1, didx1, rows1, sem1))

    pltpu.sync_copy(src_hbm.at[pl.ds(ebase, CH)], sidx0)
    pltpu.sync_copy(dst_hbm.at[pl.ds(ebase, CH)], didx0)
    pltpu.async_copy(x_hbm.at[sidx0], rows0, sem0)

    @pl.loop(0, NCH, step=2)
    def _chunk(c):
        for b in range(2):
            cc = c + b
            csi, cdi, crw, csem = bufs[b]
            nsi, ndi, nrw, nsem = bufs[1 - b]

            @pl.when(cc < NCH)
            def _do():
                @pl.when(cc + 1 < NCH)
                def _prefetch():
                    base1 = ebase + (cc + 1) * CH
                    pltpu.sync_copy(src_hbm.at[pl.ds(base1, CH)], nsi)
                    pltpu.async_copy(x_hbm.at[nsi], nrw, nsem)
                    pltpu.sync_copy(dst_hbm.at[pl.ds(base1, CH)], ndi)

                pltpu.make_async_copy(x_hbm.at[csi], crw, csem).wait()
                pltpu.sync_copy(crw, aggS.at[cdi], add=True)

    plsc.subcore_barrier()
    for z in range(RPT // CH):
        zbase = start + z * CH
        for k in range(CH // LANES):
            didx0[pl.ds(LANES * k, LANES)] = rowid16 + (zbase + LANES * k)
        pltpu.async_copy(aggS.at[didx0], rows0, sem0).wait()
        pltpu.sync_copy(rows0, agg_out.at[cid, pl.ds(zbase, CH)])


@functools.partial(
    pl.kernel,
    out_type=jax.ShapeDtypeStruct((E * LANES,), jnp.float32),
    mesh=_mesh(),
    scratch_types=[
        pltpu.VMEM((CH,), jnp.int32),
        pltpu.VMEM((CH,), jnp.int32),
        pltpu.VMEM((CH, D), jnp.float32),
        pltpu.VMEM((CH, D), jnp.float32),
        pltpu.VMEM((CH * LANES,), jnp.float32),
        pltpu.VMEM((D,), jnp.float32),
        pltpu.SemaphoreType.DMA,
        pltpu.SemaphoreType.DMA,
    ],
)
def _sc_edge(a_hbm, b_hbm, src_hbm, dst_hbm, w_hbm, out_hbm,
             sidx, didx, rowsA, rowsB, accb, wbuf, semA, semB):
    cid = lax.axis_index("c")
    sid = lax.axis_index("s")
    wid = cid * NS + sid

    pltpu.sync_copy(w_hbm, wbuf)
    wv = [wbuf[pl.ds(LANES * j, LANES)] for j in range(D // LANES)]

    ebase = wid * EPT

    @pl.loop(0, NCH)
    def _chunk(c):
        base = ebase + c * CH
        pltpu.sync_copy(src_hbm.at[pl.ds(base, CH)], sidx)
        pltpu.sync_copy(dst_hbm.at[pl.ds(base, CH)], didx)
        cpa = pltpu.async_copy(a_hbm.at[sidx], rowsA, semA)
        cpb = pltpu.async_copy(b_hbm.at[didx], rowsB, semB)
        cpa.wait()
        cpb.wait()

        @pl.loop(0, CH)
        def _edge(e):
            acc = None
            for j in range(D // LANES):
                va = rowsA[e, pl.ds(LANES * j, LANES)]
                vb = rowsB[e, pl.ds(LANES * j, LANES)]
                t = jnp.maximum(va + vb, 0.0) * wv[j]
                acc = t if acc is None else acc + t
            accb[pl.ds(e * LANES, LANES)] = acc

        pltpu.sync_copy(accb, out_hbm.at[pl.ds(base * LANES, CH * LANES)])


_BN = 1000
_PREC = lax.Precision.HIGHEST


def _tc_layer1_body(aggp, cntp, x, w1l, b1l, w1r, h1_out):
    cnt = cntp[0, :, 0:1] + cntp[1, :, 0:1]
    inv = 1.0 / jnp.maximum(cnt, 1.0)
    mean1 = (aggp[0] + aggp[1]) * inv
    z = (
        jnp.dot(mean1, w1l[...], preferred_element_type=jnp.float32, precision=_PREC)
        + b1l[...]
        + jnp.dot(x[...], w1r[...], preferred_element_type=jnp.float32, precision=_PREC)
    )
    h1_out[...] = jnp.maximum(z, 0.0)


def _tc_layer1(aggp, cntp, x, w1l, b1l, w1r):
    grid = N // _BN
    return pl.pallas_call(
        _tc_layer1_body,
        grid=(grid,),
        in_specs=[
            pl.BlockSpec((NC, _BN, D), lambda i: (0, i, 0)),
            pl.BlockSpec((NC, _BN, D), lambda i: (0, i, 0)),
            pl.BlockSpec((_BN, D), lambda i: (i, 0)),
            pl.BlockSpec((D, H), lambda i: (0, 0)),
            pl.BlockSpec((1, H), lambda i: (0, 0)),
            pl.BlockSpec((D, H), lambda i: (0, 0)),
        ],
        out_specs=pl.BlockSpec((_BN, H), lambda i: (i, 0)),
        out_shape=jax.ShapeDtypeStruct((N, H), jnp.float32),
    )(aggp, cntp, x, w1l, b1l, w1r)


def _tc_layer2_body(aggp, cntp, h1, w2l, b2l, w2r, wm1t, wm1b, bm1, a_out, b_out):
    cnt = cntp[0, :, 0:1] + cntp[1, :, 0:1]
    inv = 1.0 / jnp.maximum(cnt, 1.0)
    mean2 = (aggp[0] + aggp[1]) * inv
    z = (
        jnp.dot(mean2, w2l[...], preferred_element_type=jnp.float32, precision=_PREC)
        + b2l[...]
        + jnp.dot(h1[...], w2r[...], preferred_element_type=jnp.float32, precision=_PREC)
    )
    h2 = jnp.maximum(z, 0.0)
    a_out[...] = (
        jnp.dot(h2, wm1t[...], preferred_element_type=jnp.float32, precision=_PREC)
        + bm1[...]
    )
    b_out[...] = jnp.dot(h2, wm1b[...], preferred_element_type=jnp.float32, precision=_PREC)


def _tc_layer2(aggp, cntp, h1, w2l, b2l, w2r, wm1t, wm1b, bm1):
    grid = N // _BN
    return pl.pallas_call(
        _tc_layer2_body,
        grid=(grid,),
        in_specs=[
            pl.BlockSpec((NC, _BN, D), lambda i: (0, i, 0)),
            pl.BlockSpec((NC, _BN, D), lambda i: (0, i, 0)),
            pl.BlockSpec((_BN, H), lambda i: (i, 0)),
            pl.BlockSpec((H, H), lambda i: (0, 0)),
            pl.BlockSpec((1, H), lambda i: (0, 0)),
            pl.BlockSpec((H, H), lambda i: (0, 0)),
            pl.BlockSpec((H, H), lambda i: (0, 0)),
            pl.BlockSpec((H, H), lambda i: (0, 0)),
            pl.BlockSpec((1, H), lambda i: (0, 0)),
        ],
        out_specs=[
            pl.BlockSpec((_BN, H), lambda i: (i, 0)),
            pl.BlockSpec((_BN, H), lambda i: (i, 0)),
        ],
        out_shape=[
            jax.ShapeDtypeStruct((N, H), jnp.float32),
            jax.ShapeDtypeStruct((N, H), jnp.float32),
        ],
    )(aggp, cntp, h1, w2l, b2l, w2r, wm1t, wm1b, bm1)


def _jnp_sage(x, src, dst, Wl, bl, Wr):
    msg = jnp.take(x, src, axis=0)
    agg = jax.ops.segment_sum(msg, dst, num_segments=N)
    cnt = jax.ops.segment_sum(jnp.ones((src.shape[0],), x.dtype), dst, num_segments=N)
    mean = agg / jnp.clip(cnt, 1.0)[:, None]
    return mean @ Wl + bl + x @ Wr


_PR = E * LANES // 128
_BPR = 4000


def _tc_final_body(p, g, bm2, out):
    s = jnp.dot(p[...], g[...], preferred_element_type=jnp.float32, precision=_PREC)
    out[...] = jax.nn.sigmoid(s + bm2[...])


def _tc_final(p128, g, bm2):
    grid = _PR // _BPR
    return pl.pallas_call(
        _tc_final_body,
        grid=(grid,),
        in_specs=[
            pl.BlockSpec((_BPR, 128), lambda i: (i, 0)),
            pl.BlockSpec((128, 8), lambda i: (0, 0)),
            pl.BlockSpec((1, 1), lambda i: (0, 0)),
        ],
        out_specs=pl.BlockSpec((_BPR, 8), lambda i: (i, 0)),
        out_shape=jax.ShapeDtypeStruct((_PR, 8), jnp.float32),
    )(p128, g, bm2)


def kernel(x, edge_index, W1l, b1l, W1r, W2l, b2l, W2r, Wm1, bm1, Wm2, bm2):
    src = edge_index[0]
    dst = edge_index[1]
    zeros_c = jnp.zeros((CH, D), jnp.float32)
    ones_c = jnp.ones((CH, D), jnp.float32)
    cntp = _sc_cnt(dst, zeros_c, ones_c)
    aggp = _sc_agg(x, src, dst, zeros_c)
    h1 = _tc_layer1(aggp, cntp, x, W1l, b1l.reshape(1, H), W1r)
    agg2p = _sc_agg(h1, src, dst, zeros_c)
    a_nodes, b_nodes = _tc_layer2(
        agg2p, cntp, h1, W2l, b2l.reshape(1, H), W2r,
        Wm1[:H], Wm1[H:], bm1.reshape(1, H),
    )
    p_flat = _sc_edge(a_nodes, b_nodes, src, dst, Wm2.reshape(H))
    g = (jnp.arange(128)[:, None] // LANES == jnp.arange(8)[None, :]).astype(jnp.float32)
    out8 = _tc_final(p_flat.reshape(_PR, 128), g, bm2.reshape(1, 1))
    return out8.reshape(E)


# speedup vs baseline: 5.9331x; 1.1738x over previous
import functools

import jax
import jax.numpy as jnp
from jax import lax
from jax.experimental import pallas as pl
from jax.experimental.pallas import tpu as pltpu
from jax.experimental.pallas import tpu_sc as plsc

N = 10000
E = 320000
D = 128
H = 128

NC = 2
NS = 16
NW = NC * NS
CH = 80
EPT = E // NW
NCH = EPT // CH
NP = 10240
RPT = NP // NS
LANES = 16


def _mesh():
    return plsc.VectorSubcoreMesh(
        core_axis_name="c", subcore_axis_name="s", num_cores=NC, num_subcores=NS
    )


@functools.partial(
    pl.kernel,
    out_type=jax.ShapeDtypeStruct((NC, NP, D), jnp.float32),
    mesh=_mesh(),
    scratch_types=[
        pltpu.VMEM((CH,), jnp.int32),
        pltpu.VMEM((CH, D), jnp.float32),
        pltpu.VMEM((CH, D), jnp.float32),
        pltpu.VMEM_SHARED((NP, D), jnp.float32),
        pltpu.SemaphoreType.DMA,
    ],
)
def _sc_cnt(dst_hbm, zeros_hbm, ones_hbm, cnt_out, didx, rows, ones, cntS, sem):
    cid = lax.axis_index("c")
    sid = lax.axis_index("s")
    wid = cid * NS + sid

    rowid16 = lax.iota(jnp.int32, LANES)
    pltpu.sync_copy(zeros_hbm, rows)
    pltpu.sync_copy(ones_hbm, ones)

    start = sid * RPT
    for z in range(RPT // CH):
        zbase = start + z * CH
        for k in range(CH // LANES):
            didx[pl.ds(LANES * k, LANES)] = rowid16 + (zbase + LANES * k)
        pltpu.sync_copy(rows, cntS.at[didx])
    plsc.subcore_barrier()

    ebase = wid * EPT

    @pl.loop(0, NCH)
    def _chunk(c):
        base = ebase + c * CH
        pltpu.sync_copy(dst_hbm.at[pl.ds(base, CH)], didx)
        pltpu.sync_copy(ones, cntS.at[didx], add=True)

    plsc.subcore_barrier()
    for z in range(RPT // CH):
        zbase = start + z * CH
        for k in range(CH // LANES):
            didx[pl.ds(LANES * k, LANES)] = rowid16 + (zbase + LANES * k)
        pltpu.async_copy(cntS.at[didx], rows, sem).wait()
        pltpu.sync_copy(rows, cnt_out.at[cid, pl.ds(zbase, CH)])


@functools.partial(
    pl.kernel,
    out_type=jax.ShapeDtypeStruct((NC, NP, D), jnp.float32),
    mesh=_mesh(),
    scratch_types=[
        pltpu.VMEM((CH,), jnp.int32),
        pltpu.VMEM((CH,), jnp.int32),
        pltpu.VMEM((CH,), jnp.int32),
        pltpu.VMEM((CH,), jnp.int32),
        pltpu.VMEM((CH, D), jnp.float32),
        pltpu.VMEM((CH, D), jnp.float32),
        pltpu.VMEM_SHARED((NP, D), jnp.float32),
        pltpu.SemaphoreType.DMA,
        pltpu.SemaphoreType.DMA,
    ],
)
def _sc_agg(x_hbm, src_hbm, dst_hbm, zeros_hbm, agg_out,
            sidx0, didx0, sidx1, didx1, rows0, rows1, aggS, sem0, sem1):
    cid = lax.axis_index("c")
    sid = lax.axis_index("s")
    wid = cid * NS + sid

    rowid16 = lax.iota(jnp.int32, LANES)

    pltpu.sync_copy(zeros_hbm, rows0)

    start = sid * RPT
    for z in range(RPT // CH):
        zbase = start + z * CH
        for k in range(CH // LANES):
            didx0[pl.ds(LANES * k, LANES)] = rowid16 + (zbase + LANES * k)
        pltpu.sync_copy(rows0, aggS.at[didx0])
    plsc.subcore_barrier()

    ebase = wid * EPT
    bufs = ((sidx0, didx0, rows0, sem0), (sidx1, didx1, rows1, sem1))

    pltpu.sync_copy(src_hbm.at[pl.ds(ebase, CH)], sidx0)
    pltpu.sync_copy(dst_hbm.at[pl.ds(ebase, CH)], didx0)
    pltpu.async_copy(x_hbm.at[sidx0], rows0, sem0)

    @pl.loop(0, NCH, step=2)
    def _chunk(c):
        for b in range(2):
            cc = c + b
            csi, cdi, crw, csem = bufs[b]
            nsi, ndi, nrw, nsem = bufs[1 - b]

            @pl.when(cc < NCH)
            def _do():
                @pl.when(cc + 1 < NCH)
                def _prefetch():
                    base1 = ebase + (cc + 1) * CH
                    pltpu.sync_copy(src_hbm.at[pl.ds(base1, CH)], nsi)
                    pltpu.async_copy(x_hbm.at[nsi], nrw, nsem)
                    pltpu.sync_copy(dst_hbm.at[pl.ds(base1, CH)], ndi)

                pltpu.make_async_copy(x_hbm.at[csi], crw, csem).wait()
                pltpu.sync_copy(crw, aggS.at[cdi], add=True)

    plsc.subcore_barrier()
    for z in range(RPT // CH):
        zbase = start + z * CH
        for k in range(CH // LANES):
            didx0[pl.ds(LANES * k, LANES)] = rowid16 + (zbase + LANES * k)
        pltpu.async_copy(aggS.at[didx0], rows0, sem0).wait()
        pltpu.sync_copy(rows0, agg_out.at[cid, pl.ds(zbase, CH)])


@functools.partial(
    pl.kernel,
    out_type=jax.ShapeDtypeStruct((E * LANES,), jnp.float32),
    mesh=_mesh(),
    scratch_types=[
        pltpu.VMEM((CH,), jnp.int32),
        pltpu.VMEM((CH,), jnp.int32),
        pltpu.VMEM((CH,), jnp.int32),
        pltpu.VMEM((CH,), jnp.int32),
        pltpu.VMEM((CH, D), jnp.float32),
        pltpu.VMEM((CH, D), jnp.float32),
        pltpu.VMEM((CH, D), jnp.float32),
        pltpu.VMEM((CH, D), jnp.float32),
        pltpu.VMEM((CH * LANES,), jnp.float32),
        pltpu.VMEM((D,), jnp.float32),
        pltpu.SemaphoreType.DMA,
        pltpu.SemaphoreType.DMA,
        pltpu.SemaphoreType.DMA,
        pltpu.SemaphoreType.DMA,
    ],
)
def _sc_edge(a_hbm, b_hbm, src_hbm, dst_hbm, w_hbm, out_hbm,
             sidx0, didx0, sidx1, didx1, rowsA0, rowsB0, rowsA1, rowsB1,
             accb, wbuf, semA0, semB0, semA1, semB1):
    cid = lax.axis_index("c")
    sid = lax.axis_index("s")
    wid = cid * NS + sid

    pltpu.sync_copy(w_hbm, wbuf)
    wv = [wbuf[pl.ds(LANES * j, LANES)] for j in range(D // LANES)]

    ebase = wid * EPT
    bufs = ((sidx0, didx0, rowsA0, rowsB0, semA0, semB0),
            (sidx1, didx1, rowsA1, rowsB1, semA1, semB1))

    pltpu.sync_copy(src_hbm.at[pl.ds(ebase, CH)], sidx0)
    pltpu.sync_copy(dst_hbm.at[pl.ds(ebase, CH)], didx0)
    pltpu.async_copy(a_hbm.at[sidx0], rowsA0, semA0)
    pltpu.async_copy(b_hbm.at[didx0], rowsB0, semB0)

    @pl.loop(0, NCH, step=2)
    def _chunk(c):
        for b in range(2):
            cc = c + b
            csi, cdi, cra, crb, csa, csb = bufs[b]
            nsi, ndi, nra, nrb, nsa, nsb = bufs[1 - b]

            @pl.when(cc < NCH)
            def _do():
                @pl.when(cc + 1 < NCH)
                def _prefetch():
                    base1 = ebase + (cc + 1) * CH
                    pltpu.sync_copy(src_hbm.at[pl.ds(base1, CH)], nsi)
                    pltpu.async_copy(a_hbm.at[nsi], nra, nsa)
                    pltpu.sync_copy(dst_hbm.at[pl.ds(base1, CH)], ndi)
                    pltpu.async_copy(b_hbm.at[ndi], nrb, nsb)

                pltpu.make_async_copy(a_hbm.at[csi], cra, csa).wait()
                pltpu.make_async_copy(b_hbm.at[cdi], crb, csb).wait()

                @pl.loop(0, CH)
                def _edge(e):
                    acc = None
                    for j in range(D // LANES):
                        va = cra[e, pl.ds(LANES * j, LANES)]
                        vb = crb[e, pl.ds(LANES * j, LANES)]
                        t = jnp.maximum(va + vb, 0.0) * wv[j]
                        acc = t if acc is None else acc + t
                    accb[pl.ds(e * LANES, LANES)] = acc

                base = ebase + cc * CH
                pltpu.sync_copy(accb, out_hbm.at[pl.ds(base * LANES, CH * LANES)])


_BN = 1000
_PREC = lax.Precision.HIGHEST


def _tc_layer1_body(aggp, cntp, x, w1l, b1l, w1r, h1_out):
    cnt = cntp[0, :, 0:1] + cntp[1, :, 0:1]
    inv = 1.0 / jnp.maximum(cnt, 1.0)
    mean1 = (aggp[0] + aggp[1]) * inv
    z = (
        jnp.dot(mean1, w1l[...], preferred_element_type=jnp.float32, precision=_PREC)
        + b1l[...]
        + jnp.dot(x[...], w1r[...], preferred_element_type=jnp.float32, precision=_PREC)
    )
    h1_out[...] = jnp.maximum(z, 0.0)


def _tc_layer1(aggp, cntp, x, w1l, b1l, w1r):
    grid = N // _BN
    return pl.pallas_call(
        _tc_layer1_body,
        grid=(grid,),
        in_specs=[
            pl.BlockSpec((NC, _BN, D), lambda i: (0, i, 0)),
            pl.BlockSpec((NC, _BN, D), lambda i: (0, i, 0)),
            pl.BlockSpec((_BN, D), lambda i: (i, 0)),
            pl.BlockSpec((D, H), lambda i: (0, 0)),
            pl.BlockSpec((1, H), lambda i: (0, 0)),
            pl.BlockSpec((D, H), lambda i: (0, 0)),
        ],
        out_specs=pl.BlockSpec((_BN, H), lambda i: (i, 0)),
        out_shape=jax.ShapeDtypeStruct((N, H), jnp.float32),
    )(aggp, cntp, x, w1l, b1l, w1r)


def _tc_layer2_body(aggp, cntp, h1, w2l, b2l, w2r, wm1t, wm1b, bm1, a_out, b_out):
    cnt = cntp[0, :, 0:1] + cntp[1, :, 0:1]
    inv = 1.0 / jnp.maximum(cnt, 1.0)
    mean2 = (aggp[0] + aggp[1]) * inv
    z = (
        jnp.dot(mean2, w2l[...], preferred_element_type=jnp.float32, precision=_PREC)
        + b2l[...]
        + jnp.dot(h1[...], w2r[...], preferred_element_type=jnp.float32, precision=_PREC)
    )
    h2 = jnp.maximum(z, 0.0)
    a_out[...] = (
        jnp.dot(h2, wm1t[...], preferred_element_type=jnp.float32, precision=_PREC)
        + bm1[...]
    )
    b_out[...] = jnp.dot(h2, wm1b[...], preferred_element_type=jnp.float32, precision=_PREC)


def _tc_layer2(aggp, cntp, h1, w2l, b2l, w2r, wm1t, wm1b, bm1):
    grid = N // _BN
    return pl.pallas_call(
        _tc_layer2_body,
        grid=(grid,),
        in_specs=[
            pl.BlockSpec((NC, _BN, D), lambda i: (0, i, 0)),
            pl.BlockSpec((NC, _BN, D), lambda i: (0, i, 0)),
            pl.BlockSpec((_BN, H), lambda i: (i, 0)),
            pl.BlockSpec((H, H), lambda i: (0, 0)),
            pl.BlockSpec((1, H), lambda i: (0, 0)),
            pl.BlockSpec((H, H), lambda i: (0, 0)),
            pl.BlockSpec((H, H), lambda i: (0, 0)),
            pl.BlockSpec((H, H), lambda i: (0, 0)),
            pl.BlockSpec((1, H), lambda i: (0, 0)),
        ],
        out_specs=[
            pl.BlockSpec((_BN, H), lambda i: (i, 0)),
            pl.BlockSpec((_BN, H), lambda i: (i, 0)),
        ],
        out_shape=[
            jax.ShapeDtypeStruct((N, H), jnp.float32),
            jax.ShapeDtypeStruct((N, H), jnp.float32),
        ],
    )(aggp, cntp, h1, w2l, b2l, w2r, wm1t, wm1b, bm1)


def _jnp_sage(x, src, dst, Wl, bl, Wr):
    msg = jnp.take(x, src, axis=0)
    agg = jax.ops.segment_sum(msg, dst, num_segments=N)
    cnt = jax.ops.segment_sum(jnp.ones((src.shape[0],), x.dtype), dst, num_segments=N)
    mean = agg / jnp.clip(cnt, 1.0)[:, None]
    return mean @ Wl + bl + x @ Wr


_PR = E * LANES // 128
_BPR = 4000


def _tc_final_body(p, g, bm2, out):
    s = jnp.dot(p[...], g[...], preferred_element_type=jnp.float32, precision=_PREC)
    out[...] = jax.nn.sigmoid(s + bm2[...])


def _tc_final(p128, g, bm2):
    grid = _PR // _BPR
    return pl.pallas_call(
        _tc_final_body,
        grid=(grid,),
        in_specs=[
            pl.BlockSpec((_BPR, 128), lambda i: (i, 0)),
            pl.BlockSpec((128, 8), lambda i: (0, 0)),
            pl.BlockSpec((1, 1), lambda i: (0, 0)),
        ],
        out_specs=pl.BlockSpec((_BPR, 8), lambda i: (i, 0)),
        out_shape=jax.ShapeDtypeStruct((_PR, 8), jnp.float32),
    )(p128, g, bm2)


def kernel(x, edge_index, W1l, b1l, W1r, W2l, b2l, W2r, Wm1, bm1, Wm2, bm2):
    src = edge_index[0]
    dst = edge_index[1]
    zeros_c = jnp.zeros((CH, D), jnp.float32)
    ones_c = jnp.ones((CH, D), jnp.float32)
    cntp = _sc_cnt(dst, zeros_c, ones_c)
    aggp = _sc_agg(x, src, dst, zeros_c)
    h1 = _tc_layer1(aggp, cntp, x, W1l, b1l.reshape(1, H), W1r)
    agg2p = _sc_agg(h1, src, dst, zeros_c)
    a_nodes, b_nodes = _tc_layer2(
        agg2p, cntp, h1, W2l, b2l.reshape(1, H), W2r,
        Wm1[:H], Wm1[H:], bm1.reshape(1, H),
    )
    p_flat = _sc_edge(a_nodes, b_nodes, src, dst, Wm2.reshape(H))
    g = (jnp.arange(128)[:, None] // LANES == jnp.arange(8)[None, :]).astype(jnp.float32)
    out8 = _tc_final(p_flat.reshape(_PR, 128), g, bm2.reshape(1, 1))
    return out8.reshape(E)
